# Initial kernel scaffold; baseline (speedup 1.0000x reference)
import jax
import jax.numpy as jnp
from jax.experimental import pallas as pl


def kernel(edge_index, h, e, Qw, Qb, Kw, Kb, Vw, Vb, Ew, Eb):
    raise NotImplementedError("write your pallas kernel here")


# speedup vs baseline: 37.0970x; 37.0970x over previous
import functools

import jax
import jax.numpy as jnp
import numpy as np
from jax import lax
from jax.experimental import pallas as pl
from jax.experimental.pallas import tpu as pltpu
from jax.experimental.pallas import tpu_sc as plsc

N_NODES = 10000
N_EDGES = 320000
D = 128
H = 8
HD = 16

NW = 32
EPW = N_EDGES // NW
B = 80
NCHUNK = EPW // B
ROWS_PER_SUB = N_NODES // 16


def _matmul_bias_body(x_ref, w_ref, b_ref, o_ref):
    o_ref[...] = (
        jnp.dot(x_ref[...], w_ref[...], preferred_element_type=jnp.float32)
        + b_ref[...]
    )


def _matmul_bias(x, w, b, block_rows):
    rows, kdim = x.shape
    cols = w.shape[1]
    grid = rows // block_rows
    return pl.pallas_call(
        _matmul_bias_body,
        grid=(grid,),
        in_specs=[
            pl.BlockSpec((block_rows, kdim), lambda i: (i, 0)),
            pl.BlockSpec((kdim, cols), lambda i: (0, 0)),
            pl.BlockSpec((1, cols), lambda i: (0, 0)),
        ],
        out_specs=pl.BlockSpec((block_rows, cols), lambda i: (i, 0)),
        out_shape=jax.ShapeDtypeStruct((rows, cols), jnp.float32),
    )(x, w, b.reshape(1, cols))


def _sc_body(src_hbm, dst_hbm, kh_hbm, qh_hbm, pe_hbm,
             eout_hbm, zout_hbm,
             sidx, didx, kbuf, qbuf, pebuf, ebuf, zbuf, zrows, zsh,
             semk, semq, seme):
    cid = lax.axis_index("c")
    sid = lax.axis_index("s")
    wid = cid * 16 + sid

    def zrow_body(i, _):
        zrows[i, :] = jnp.zeros((16,), jnp.float32)
        return 0
    lax.fori_loop(0, 125, zrow_body, 0)

    def zcopy_body(j, _):
        pltpu.sync_copy(zrows, zsh.at[pl.ds(sid * ROWS_PER_SUB + j * 125, 125)])
        return 0
    lax.fori_loop(0, 5, zcopy_body, 0)

    plsc.subcore_barrier()

    lane = lax.iota(jnp.int32, 16)
    onehot = [
        jnp.where(lane == hh, 1.0, 0.0).astype(jnp.float32) for hh in range(H)
    ]

    def chunk_body(c, _):
        base = wid * EPW + c * B
        pltpu.sync_copy(src_hbm.at[pl.ds(base, B)], sidx)
        pltpu.sync_copy(dst_hbm.at[pl.ds(base, B)], didx)
        cpk = pltpu.async_copy(kh_hbm.at[sidx], kbuf, semk)
        cpq = pltpu.async_copy(qh_hbm.at[didx], qbuf, semq)
        cpe = pltpu.async_copy(pe_hbm.at[pl.ds(base, B), :], pebuf, seme)
        cpk.wait()
        cpq.wait()
        cpe.wait()

        def edge_body(i, _):
            row = jnp.zeros((16,), jnp.float32)
            for hh in range(H):
                sl = pl.ds(hh * HD, HD)
                s = kbuf[i, sl] * qbuf[i, sl] * 0.25 + pebuf[i, sl]
                ebuf[i, sl] = s
                row = row + jnp.full((16,), jnp.sum(s)) * onehot[hh]
            zbuf[i, :] = jnp.exp(jnp.clip(row, -5.0, 5.0))
            return 0
        lax.fori_loop(0, B, edge_body, 0)

        pltpu.sync_copy(ebuf, eout_hbm.at[pl.ds(base, B), :])
        pltpu.sync_copy(zbuf, zsh.at[sidx], add=True)
        return 0

    lax.fori_loop(0, NCHUNK, chunk_body, 0)

    plsc.subcore_barrier()

    @pl.when(sid == 0)
    def _():
        pltpu.sync_copy(zsh, zout_hbm.at[cid])


def _sc_call(src, dst, kh, qh, pe):
    mesh = plsc.VectorSubcoreMesh(
        core_axis_name="c", subcore_axis_name="s", num_cores=2, num_subcores=16)
    return pl.kernel(
        _sc_body,
        out_type=[
            jax.ShapeDtypeStruct((N_EDGES, D), jnp.float32),
            jax.ShapeDtypeStruct((2, N_NODES, 16), jnp.float32),
        ],
        mesh=mesh,
        compiler_params=pltpu.CompilerParams(
            needs_layout_passes=False, use_tc_tiling_on_sc=False),
        scratch_types=[
            pltpu.VMEM((B,), jnp.int32),
            pltpu.VMEM((B,), jnp.int32),
            pltpu.VMEM((B, D), jnp.float32),
            pltpu.VMEM((B, D), jnp.float32),
            pltpu.VMEM((B, D), jnp.float32),
            pltpu.VMEM((B, D), jnp.float32),
            pltpu.VMEM((B, 16), jnp.float32),
            pltpu.VMEM((125, 16), jnp.float32),
            pltpu.VMEM_SHARED((N_NODES, 16), jnp.float32),
            pltpu.SemaphoreType.DMA,
            pltpu.SemaphoreType.DMA,
            pltpu.SemaphoreType.DMA,
        ],
    )(src, dst, kh, qh, pe)


def _finish_body(v_ref, zp_ref, sz_ref, sc_ref, o_ref):
    zs = zp_ref[0] + zp_ref[1]
    zfull = jnp.dot(zs, sz_ref[...], preferred_element_type=jnp.float32)
    cfull = jnp.dot(zs, sc_ref[...], preferred_element_type=jnp.float32)
    z = jnp.where(cfull > 0.0, zfull / jnp.maximum(cfull, 1.0), 0.0)
    o_ref[...] = v_ref[...] * zfull / (z + 1e-6)


def _finish(v_h, zpart, sz, sc):
    block = 1000
    grid = N_NODES // block
    return pl.pallas_call(
        _finish_body,
        grid=(grid,),
        in_specs=[
            pl.BlockSpec((block, D), lambda i: (i, 0)),
            pl.BlockSpec((2, block, 16), lambda i: (0, i, 0)),
            pl.BlockSpec((16, D), lambda i: (0, 0)),
            pl.BlockSpec((16, D), lambda i: (0, 0)),
        ],
        out_specs=pl.BlockSpec((block, D), lambda i: (i, 0)),
        out_shape=jax.ShapeDtypeStruct((N_NODES, D), jnp.float32),
    )(v_h, zpart, sz, sc)


_SZ = np.zeros((16, D), np.float32)
for _h in range(H):
    _SZ[_h, _h * HD:(_h + 1) * HD] = 1.0
_SC = np.zeros((16, D), np.float32)
_SC[8, :] = 1.0


def kernel(edge_index, h, e, Qw, Qb, Kw, Kb, Vw, Vb, Ew, Eb):
    src = edge_index[0].astype(jnp.int32)
    dst = edge_index[1].astype(jnp.int32)

    wqkv = jnp.concatenate([Qw, Kw, Vw], axis=1)
    bqkv = jnp.concatenate([Qb, Kb, Vb], axis=0)
    qkv = _matmul_bias(h, wqkv, bqkv, block_rows=1000)
    q_h = qkv[:, 0:D]
    k_h = qkv[:, D:2 * D]
    v_h = qkv[:, 2 * D:3 * D]
    pe = _matmul_bias(e, Ew, Eb, block_rows=8000)

    e_out, zpart = _sc_call(src, dst, k_h, qh=q_h, pe=pe)

    h_out = _finish(v_h, zpart, jnp.asarray(_SZ), jnp.asarray(_SC))

    return (h_out.reshape(N_NODES, H, HD), e_out.reshape(N_EDGES, H, HD))
